# Initial kernel scaffold; baseline (speedup 1.0000x reference)
import jax
import jax.numpy as jnp
from jax.experimental import pallas as pl


def kernel(x, params):
    raise NotImplementedError("write your pallas kernel here")


# speedup vs baseline: 1.4123x; 1.4123x over previous
import functools

import jax
import jax.numpy as jnp
import numpy as np
from jax.experimental import pallas as pl
from jax.experimental.pallas import tpu as pltpu

UP = 4
CAPS = (34, 38, 29)
D = 56
S = 12


def _classifier_body(x48_ref, w1_ref, b1_ref, w2_ref, b2_ref, w3_ref, b3_ref,
                     w4_ref, b4_ref, w5_ref, b5_ref, fcw_ref, fcb_ref,
                     logits_ref):
    def lrelu(h):
        return jnp.where(h >= 0, h, 0.1 * h)

    h = jnp.dot(x48_ref[...], w1_ref[...],
                preferred_element_type=jnp.float32) + b1_ref[...]
    h = lrelu(h)
    h = lrelu(jnp.dot(h, w2_ref[...],
                      preferred_element_type=jnp.float32) + b2_ref[...])
    h = lrelu(jnp.dot(h, w3_ref[...],
                      preferred_element_type=jnp.float32) + b3_ref[...])
    h = lrelu(jnp.dot(h, w4_ref[...],
                      preferred_element_type=jnp.float32) + b4_ref[...])
    h = jnp.dot(h, w5_ref[...],
                preferred_element_type=jnp.float32) + b5_ref[...]
    hm = jnp.mean(h.reshape(64, 64, 32), axis=1)
    logits_ref[...] = jnp.dot(hm, fcw_ref[...],
                              preferred_element_type=jnp.float32) + fcb_ref[...]


def _run_classifier(x, cls):
    x48 = x.reshape(64, 3, 8, 4, 8, 4).transpose(0, 2, 4, 3, 5, 1).reshape(4096, 48)
    w1 = cls['w1'].transpose(2, 3, 1, 0).reshape(48, 128)
    fcw = cls['fc_w'].T
    args = (x48, w1, cls['b1'][None, :], cls['w2'][:, :, 0, 0].T,
            cls['b2'][None, :], cls['w3'][:, :, 0, 0].T, cls['b3'][None, :],
            cls['w4'][:, :, 0, 0].T, cls['b4'][None, :],
            cls['w5'][:, :, 0, 0].T, cls['b5'][None, :], fcw,
            cls['fc_b'][None, :])
    return pl.pallas_call(
        _classifier_body,
        out_shape=jax.ShapeDtypeStruct((64, 3), jnp.float32),
    )(*args)


def _im2col3(h, C):
    h3 = h.reshape(32, 32, C)
    hp = jnp.pad(h3, ((1, 1), (1, 1), (0, 0)))
    cols = [hp[dy:dy + 32, dx:dx + 32, :].reshape(1024, C)
            for dy in range(3) for dx in range(3)]
    return jnp.concatenate(cols, axis=1)


def _fsrcnn_body(sel_ref, val_ref, x3_ref, wh_ref, bh_ref, ah_ref,
                 ws_ref, bs_ref, as_ref, wm_ref, bm_ref, am_ref,
                 we_ref, be_ref, ae_ref, wd_ref, bd_ref, out_ref):
    i = pl.program_id(0)

    x3 = x3_ref[0].reshape(32, 32, 3)
    xp = jnp.pad(x3, ((2, 2), (2, 2), (0, 0)))
    cols = [xp[dy:dy + 32, dx:dx + 32, :].reshape(1024, 3)
            for dy in range(5) for dx in range(5)]
    x75 = jnp.concatenate(cols, axis=1)

    h = jnp.dot(x75, wh_ref[0], preferred_element_type=jnp.float32) + bh_ref[0]
    h = jnp.where(h >= 0, h, ah_ref[0] * h)

    h = jnp.dot(h, ws_ref[0], preferred_element_type=jnp.float32) + bs_ref[0]
    h = jnp.where(h >= 0, h, as_ref[0] * h)

    for l in range(4):
        hc = _im2col3(h, S)
        h = jnp.dot(hc, wm_ref[0, l],
                    preferred_element_type=jnp.float32) + bm_ref[0, l]
        h = jnp.where(h >= 0, h, am_ref[0, l] * h)

    h = jnp.dot(h, we_ref[0], preferred_element_type=jnp.float32) + be_ref[0]
    h = jnp.where(h >= 0, h, ae_ref[0] * h)

    hc = _im2col3(h, D)
    y = jnp.dot(hc, wd_ref[0], preferred_element_type=jnp.float32) + bd_ref[0]
    v = val_ref[i].astype(jnp.float32)
    out_ref[0] = y * v


@functools.partial(jax.jit, static_argnums=())
def _pack_expert_params(nets):
    Wh, Bh, Ah, Ws, Bs, As = [], [], [], [], [], []
    Wm, Bm, Am, We, Be, Ae, Wd, Bd = [], [], [], [], [], [], [], []
    for p in nets:
        d = p['w_head'].shape[0]
        wh = p['w_head'].transpose(2, 3, 1, 0).reshape(75, d)
        Wh.append(jnp.pad(wh, ((0, 0), (0, D - d))))
        Bh.append(jnp.pad(p['b_head'], (0, D - d))[None])
        Ah.append(jnp.pad(p['a_head'], (0, D - d), constant_values=1.0)[None])
        ws = jnp.pad(p['w_shrink'][:, :, 0, 0].T, ((0, D - d), (0, 0)))
        Ws.append(ws)
        Bs.append(p['b_shrink'][None])
        As.append(p['a_shrink'][None])
        Wm.append(jnp.stack([w.transpose(2, 3, 1, 0).reshape(9 * S, S)
                             for w in p['w_map']]))
        Bm.append(jnp.stack([b[None] for b in p['b_map']]))
        Am.append(jnp.stack([a[None] for a in p['a_map']]))
        We.append(jnp.pad(p['w_expand'][:, :, 0, 0].T, ((0, 0), (0, D - d))))
        Be.append(jnp.pad(p['b_expand'], (0, D - d))[None])
        Ae.append(jnp.pad(p['a_expand'], (0, D - d), constant_values=1.0)[None])
        wd = jnp.zeros((9, D, 48), jnp.float32)
        wdec = p['w_deconv']
        for ry in range(4):
            for rx in range(4):
                for dy in (-1, 0, 1):
                    ky = 4 * dy + 6 - ry
                    if not 0 <= ky < 9:
                        continue
                    for dx in (-1, 0, 1):
                        kx = 4 * dx + 6 - rx
                        if not 0 <= kx < 9:
                            continue
                        tap = (dy + 1) * 3 + (dx + 1)
                        ch = (ry * 4 + rx) * 3
                        wd = wd.at[tap, :d, ch:ch + 3].set(
                            wdec[:, :, ky, kx].T)
        Wd.append(wd.reshape(9 * D, 48))
        Bd.append(jnp.tile(p['b_deconv'], 16)[None])
    return (jnp.stack(Wh), jnp.stack(Bh), jnp.stack(Ah), jnp.stack(Ws),
            jnp.stack(Bs), jnp.stack(As), jnp.stack(Wm), jnp.stack(Bm),
            jnp.stack(Am), jnp.stack(We), jnp.stack(Be), jnp.stack(Ae),
            jnp.stack(Wd), jnp.stack(Bd))


def _run_fsrcnn(x3, sel, valid, packed):
    (Wh, Bh, Ah, Ws, Bs, As, Wm, Bm, Am, We, Be, Ae, Wd, Bd) = packed

    def e_map(shape):
        zeros = (0,) * (len(shape) - 1)
        return pl.BlockSpec((1,) + shape[1:],
                            lambda i, sel_ref, val_ref, z=zeros: (sel_ref[i],) + z)

    grid_spec = pltpu.PrefetchScalarGridSpec(
        num_scalar_prefetch=2,
        grid=(64,),
        in_specs=[
            pl.BlockSpec((1, 1024, 3), lambda i, s, v: (i, 0, 0)),
            e_map(Wh.shape), e_map(Bh.shape), e_map(Ah.shape),
            e_map(Ws.shape), e_map(Bs.shape), e_map(As.shape),
            e_map(Wm.shape), e_map(Bm.shape), e_map(Am.shape),
            e_map(We.shape), e_map(Be.shape), e_map(Ae.shape),
            e_map(Wd.shape), e_map(Bd.shape),
        ],
        out_specs=pl.BlockSpec((1, 1024, 48), lambda i, s, v: (i, 0, 0)),
    )
    return pl.pallas_call(
        _fsrcnn_body,
        grid_spec=grid_spec,
        out_shape=jax.ShapeDtypeStruct((64, 1024, 48), jnp.float32),
    )(sel, valid, x3, Wh, Bh, Ah, Ws, Bs, As, Wm, Bm, Am, We, Be, Ae, Wd, Bd)


def kernel(x, params):
    logits = _run_classifier(x, params['cls'])

    expert = jnp.argmax(logits, axis=-1).astype(jnp.int32)
    onehot = (expert[:, None] == jnp.arange(3, dtype=jnp.int32)).astype(jnp.int32)
    ranks = jnp.cumsum(onehot, axis=0)
    caps = jnp.asarray(CAPS, jnp.int32)
    myrank = jnp.sum(ranks * onehot, axis=1)
    valid = (myrank <= caps[expert]).astype(jnp.int32)
    counts = jnp.minimum(ranks[-1], caps)

    packed = _pack_expert_params(params['nets'])
    x3 = x.transpose(0, 2, 3, 1).reshape(64, 1024, 3)
    y48 = _run_fsrcnn(x3, expert, valid, packed)

    y = y48.reshape(64, 32, 32, 4, 4, 3)
    y = y.transpose(0, 5, 1, 3, 2, 4).reshape(64, 3, 128, 128)
    return y, counts


# speedup vs baseline: 3.3481x; 2.3706x over previous
import jax
import jax.numpy as jnp
import numpy as np
from jax.experimental import pallas as pl
from jax.experimental.pallas import tpu as pltpu

UP = 4
CAPS = (34, 38, 29)
D = 56
S = 12


def _classifier_body(x48_ref, w1_ref, b1_ref, w2_ref, b2_ref, w3_ref, b3_ref,
                     w4_ref, b4_ref, w5_ref, b5_ref, fcw_ref, fcb_ref,
                     logits_ref):
    def lrelu(h):
        return jnp.where(h >= 0, h, 0.1 * h)

    h = jnp.dot(x48_ref[...], w1_ref[...],
                preferred_element_type=jnp.float32) + b1_ref[...]
    h = lrelu(h)
    h = lrelu(jnp.dot(h, w2_ref[...],
                      preferred_element_type=jnp.float32) + b2_ref[...])
    h = lrelu(jnp.dot(h, w3_ref[...],
                      preferred_element_type=jnp.float32) + b3_ref[...])
    h = lrelu(jnp.dot(h, w4_ref[...],
                      preferred_element_type=jnp.float32) + b4_ref[...])
    h = jnp.dot(h, w5_ref[...],
                preferred_element_type=jnp.float32) + b5_ref[...]
    hm = jnp.mean(h.reshape(64, 64, 32), axis=1)
    logits_ref[...] = jnp.dot(hm, fcw_ref[...],
                              preferred_element_type=jnp.float32) + fcb_ref[...]


def _run_classifier(x, cls):
    x48 = x.reshape(64, 3, 8, 4, 8, 4).transpose(0, 2, 4, 3, 5, 1).reshape(4096, 48)
    w1 = cls['w1'].transpose(2, 3, 1, 0).reshape(48, 128)
    fcw = cls['fc_w'].T
    args = (x48, w1, cls['b1'][None, :], cls['w2'][:, :, 0, 0].T,
            cls['b2'][None, :], cls['w3'][:, :, 0, 0].T, cls['b3'][None, :],
            cls['w4'][:, :, 0, 0].T, cls['b4'][None, :],
            cls['w5'][:, :, 0, 0].T, cls['b5'][None, :], fcw,
            cls['fc_b'][None, :])
    return pl.pallas_call(
        _classifier_body,
        out_shape=jax.ShapeDtypeStruct((64, 3), jnp.float32),
    )(*args)


def _shift_stack(h, k, masks):
    r = k // 2
    pieces = []
    t = 0
    for dy in range(-r, r + 1):
        for dx in range(-r, r + 1):
            off = dy * 32 + dx
            s = jnp.roll(h, -off, axis=1) if off else h
            pieces.append(s * masks[t][None, :])
            t += 1
    return jnp.concatenate(pieces, axis=0)


def _fsrcnn_body(sel_ref, val_ref, x3_ref, m5_ref, m3_ref,
                 wh_ref, bh_ref, ah_ref, ws_ref, bs_ref, as_ref,
                 wm_ref, bm_ref, am_ref, we_ref, be_ref, ae_ref,
                 wd_ref, bd_ref, out_ref):
    i = pl.program_id(0)
    m5 = m5_ref[...]
    m3 = m3_ref[...]

    def mm(w, im):
        return jnp.dot(w, im, preferred_element_type=jnp.float32)

    def prelu(h, b, a):
        h = h + b
        return jnp.where(h >= 0, h, a * h)

    x75 = _shift_stack(x3_ref[0], 5, m5)
    h = prelu(mm(wh_ref[0], x75), bh_ref[0], ah_ref[0])
    h = prelu(mm(ws_ref[0], h), bs_ref[0], as_ref[0])
    for l in range(4):
        im = _shift_stack(h, 3, m3)
        h = prelu(mm(wm_ref[0, l], im), bm_ref[0, l], am_ref[0, l])
    h = prelu(mm(we_ref[0], h), be_ref[0], ae_ref[0])
    im = _shift_stack(h, 3, m3)
    y = mm(wd_ref[0], im) + bd_ref[0]
    v = val_ref[i].astype(jnp.float32)
    out_ref[0] = y * v


def _make_masks():
    yy, xx = np.mgrid[0:32, 0:32]
    def mk(k):
        r = k // 2
        ms = []
        for dy in range(-r, r + 1):
            for dx in range(-r, r + 1):
                ok = ((yy + dy >= 0) & (yy + dy < 32) &
                      (xx + dx >= 0) & (xx + dx < 32))
                ms.append(ok.reshape(-1))
        return jnp.asarray(np.stack(ms).astype(np.float32))
    return mk(5), mk(3)


def _pack_expert_params(nets):
    Wh, Bh, Ah, Ws, Bs, As = [], [], [], [], [], []
    Wm, Bm, Am, We, Be, Ae, Wd, Bd = [], [], [], [], [], [], [], []
    for p in nets:
        d = p['w_head'].shape[0]
        wh = p['w_head'].transpose(0, 2, 3, 1).reshape(d, 75)
        Wh.append(jnp.pad(wh, ((0, D - d), (0, 0))))
        Bh.append(jnp.pad(p['b_head'], (0, D - d))[:, None])
        Ah.append(jnp.pad(p['a_head'], (0, D - d), constant_values=1.0)[:, None])
        Ws.append(jnp.pad(p['w_shrink'][:, :, 0, 0], ((0, 0), (0, D - d))))
        Bs.append(p['b_shrink'][:, None])
        As.append(p['a_shrink'][:, None])
        Wm.append(jnp.stack([w.transpose(0, 2, 3, 1).reshape(S, 9 * S)
                             for w in p['w_map']]))
        Bm.append(jnp.stack([b[:, None] for b in p['b_map']]))
        Am.append(jnp.stack([a[:, None] for a in p['a_map']]))
        We.append(jnp.pad(p['w_expand'][:, :, 0, 0], ((0, D - d), (0, 0))))
        Be.append(jnp.pad(p['b_expand'], (0, D - d))[:, None])
        Ae.append(jnp.pad(p['a_expand'], (0, D - d), constant_values=1.0)[:, None])
        wd = jnp.zeros((48, 9, D), jnp.float32)
        wdec = p['w_deconv']
        for ry in range(4):
            for rx in range(4):
                for dy in (-1, 0, 1):
                    ky = 4 * dy + 6 - ry
                    if not 0 <= ky < 9:
                        continue
                    for dx in (-1, 0, 1):
                        kx = 4 * dx + 6 - rx
                        if not 0 <= kx < 9:
                            continue
                        tap = (dy + 1) * 3 + (dx + 1)
                        ch = (ry * 4 + rx) * 3
                        wd = wd.at[ch:ch + 3, tap, :d].set(wdec[:, :, ky, kx])
        Wd.append(wd.reshape(48, 9 * D))
        Bd.append(jnp.tile(p['b_deconv'], 16)[:, None])
    return (jnp.stack(Wh), jnp.stack(Bh), jnp.stack(Ah), jnp.stack(Ws),
            jnp.stack(Bs), jnp.stack(As), jnp.stack(Wm), jnp.stack(Bm),
            jnp.stack(Am), jnp.stack(We), jnp.stack(Be), jnp.stack(Ae),
            jnp.stack(Wd), jnp.stack(Bd))


def _run_fsrcnn(x3, sel, valid, packed, m5, m3):
    (Wh, Bh, Ah, Ws, Bs, As, Wm, Bm, Am, We, Be, Ae, Wd, Bd) = packed

    def e_map(shape):
        zeros = (0,) * (len(shape) - 1)
        return pl.BlockSpec((1,) + shape[1:],
                            lambda i, sel_ref, val_ref, z=zeros: (sel_ref[i],) + z)

    def full(shape):
        zeros = (0,) * len(shape)
        return pl.BlockSpec(shape, lambda i, sel_ref, val_ref, z=zeros: z)

    grid_spec = pltpu.PrefetchScalarGridSpec(
        num_scalar_prefetch=2,
        grid=(64,),
        in_specs=[
            pl.BlockSpec((1, 3, 1024), lambda i, s, v: (i, 0, 0)),
            full(m5.shape), full(m3.shape),
            e_map(Wh.shape), e_map(Bh.shape), e_map(Ah.shape),
            e_map(Ws.shape), e_map(Bs.shape), e_map(As.shape),
            e_map(Wm.shape), e_map(Bm.shape), e_map(Am.shape),
            e_map(We.shape), e_map(Be.shape), e_map(Ae.shape),
            e_map(Wd.shape), e_map(Bd.shape),
        ],
        out_specs=pl.BlockSpec((1, 48, 1024), lambda i, s, v: (i, 0, 0)),
    )
    return pl.pallas_call(
        _fsrcnn_body,
        grid_spec=grid_spec,
        out_shape=jax.ShapeDtypeStruct((64, 48, 1024), jnp.float32),
    )(sel, valid, x3, m5, m3, Wh, Bh, Ah, Ws, Bs, As, Wm, Bm, Am,
      We, Be, Ae, Wd, Bd)


def kernel(x, params):
    logits = _run_classifier(x, params['cls'])

    expert = jnp.argmax(logits, axis=-1).astype(jnp.int32)
    onehot = (expert[:, None] == jnp.arange(3, dtype=jnp.int32)).astype(jnp.int32)
    ranks = jnp.cumsum(onehot, axis=0)
    caps = jnp.asarray(CAPS, jnp.int32)
    myrank = jnp.sum(ranks * onehot, axis=1)
    valid = (myrank <= caps[expert]).astype(jnp.int32)
    counts = jnp.minimum(ranks[-1], caps)

    packed = _pack_expert_params(params['nets'])
    m5, m3 = _make_masks()
    x3 = x.reshape(64, 3, 1024)
    y48 = _run_fsrcnn(x3, expert, valid, packed, m5, m3)

    y = y48.reshape(64, 4, 4, 3, 32, 32)
    y = y.transpose(0, 3, 4, 1, 5, 2).reshape(64, 3, 128, 128)
    return y, counts


# speedup vs baseline: 3.8926x; 1.1627x over previous
import jax
import jax.numpy as jnp
import numpy as np
from jax.experimental import pallas as pl
from jax.experimental.pallas import tpu as pltpu

UP = 4
CAPS = (34, 38, 29)
D = 56
S = 12


def _classifier_body(x48_ref, w1_ref, b1_ref, w2_ref, b2_ref, w3_ref, b3_ref,
                     w4_ref, b4_ref, w5_ref, b5_ref, fcw_ref, fcb_ref,
                     logits_ref):
    def lrelu(h):
        return jnp.where(h >= 0, h, 0.1 * h)

    h = jnp.dot(x48_ref[...], w1_ref[...],
                preferred_element_type=jnp.float32) + b1_ref[...]
    h = lrelu(h)
    h = lrelu(jnp.dot(h, w2_ref[...],
                      preferred_element_type=jnp.float32) + b2_ref[...])
    h = lrelu(jnp.dot(h, w3_ref[...],
                      preferred_element_type=jnp.float32) + b3_ref[...])
    h = lrelu(jnp.dot(h, w4_ref[...],
                      preferred_element_type=jnp.float32) + b4_ref[...])
    h = jnp.dot(h, w5_ref[...],
                preferred_element_type=jnp.float32) + b5_ref[...]
    hm = jnp.mean(h.reshape(64, 64, 32), axis=1)
    logits_ref[...] = jnp.dot(hm, fcw_ref[...],
                              preferred_element_type=jnp.float32) + fcb_ref[...]


def _run_classifier(x, cls):
    x48 = x.reshape(64, 3, 8, 4, 8, 4).transpose(0, 2, 4, 3, 5, 1).reshape(4096, 48)
    w1 = cls['w1'].transpose(2, 3, 1, 0).reshape(48, 128)
    fcw = cls['fc_w'].T
    args = (x48, w1, cls['b1'][None, :], cls['w2'][:, :, 0, 0].T,
            cls['b2'][None, :], cls['w3'][:, :, 0, 0].T, cls['b3'][None, :],
            cls['w4'][:, :, 0, 0].T, cls['b4'][None, :],
            cls['w5'][:, :, 0, 0].T, cls['b5'][None, :], fcw,
            cls['fc_b'][None, :])
    return pl.pallas_call(
        _classifier_body,
        out_shape=jax.ShapeDtypeStruct((64, 3), jnp.float32),
    )(*args)


def _shift_stack(h, k, masks):
    r = k // 2
    pieces = []
    t = 0
    for dy in range(-r, r + 1):
        for dx in range(-r, r + 1):
            off = dy * 32 + dx
            s = jnp.roll(h, -off, axis=1) if off else h
            pieces.append(s * masks[t][None, :])
            t += 1
    return jnp.concatenate(pieces, axis=0)


def _fsrcnn_body(sel_ref, val_ref, x3_ref, m5_ref, m3_ref,
                 wh_ref, bh_ref, ah_ref, ws_ref, bs_ref, as_ref,
                 wm_ref, bm_ref, am_ref, we_ref, be_ref, ae_ref,
                 wd_ref, bd_ref, out_ref):
    i = pl.program_id(0)
    m5 = m5_ref[...]
    m3 = m3_ref[...]

    def mm(w, im):
        return jnp.dot(w, im, preferred_element_type=jnp.float32)

    def prelu(h, b, a):
        h = h + b
        return jnp.where(h >= 0, h, a * h)

    x75 = _shift_stack(x3_ref[0], 5, m5)
    h = prelu(mm(wh_ref[0], x75), bh_ref[0], ah_ref[0])
    h = prelu(mm(ws_ref[0], h), bs_ref[0], as_ref[0])
    for l in range(4):
        im = _shift_stack(h, 3, m3)
        h = prelu(mm(wm_ref[0, l], im), bm_ref[0, l], am_ref[0, l])
    h = prelu(mm(we_ref[0], h), be_ref[0], ae_ref[0])
    im = _shift_stack(h, 3, m3)
    y = mm(wd_ref[0], im) + bd_ref[0]
    v = val_ref[i].astype(jnp.float32)
    out_ref[0] = y * v


def _make_masks():
    yy, xx = np.mgrid[0:32, 0:32]
    def mk(k):
        r = k // 2
        ms = []
        for dy in range(-r, r + 1):
            for dx in range(-r, r + 1):
                ok = ((yy + dy >= 0) & (yy + dy < 32) &
                      (xx + dx >= 0) & (xx + dx < 32))
                ms.append(ok.reshape(-1))
        return jnp.asarray(np.stack(ms).astype(np.float32))
    return mk(5), mk(3)


def _deconv_fold_idx():
    o_i = np.zeros((48, 9, D), np.int32)
    c_i = np.zeros((48, 9, D), np.int32)
    ky_i = np.full((48, 9, D), 9, np.int32)
    kx_i = np.full((48, 9, D), 9, np.int32)
    for ry in range(4):
        for rx in range(4):
            for o in range(3):
                ch = (ry * 4 + rx) * 3 + o
                o_i[ch] = o
                c_i[ch] = np.arange(D)[None, :]
                for dy in (-1, 0, 1):
                    for dx in (-1, 0, 1):
                        ky, kx = 4 * dy + 6 - ry, 4 * dx + 6 - rx
                        t = (dy + 1) * 3 + (dx + 1)
                        if 0 <= ky < 9 and 0 <= kx < 9:
                            ky_i[ch, t] = ky
                            kx_i[ch, t] = kx
    return o_i, c_i, ky_i, kx_i


_DFOLD = _deconv_fold_idx()


def _pack_expert_params(nets):
    def padd(name, axis, ones=False):
        cv = 1.0 if ones else 0.0
        outs = []
        for p in nets:
            w = p[name]
            d = w.shape[axis]
            pads = [(0, 0)] * w.ndim
            pads[axis] = (0, D - d)
            outs.append(jnp.pad(w, pads, constant_values=cv))
        return jnp.stack(outs)

    Wh = padd('w_head', 0).transpose(0, 1, 3, 4, 2).reshape(3, D, 75)
    Bh = padd('b_head', 0)[:, :, None]
    Ah = padd('a_head', 0, ones=True)[:, :, None]
    Ws = padd('w_shrink', 1)[:, :, :, 0, 0]
    Bs = jnp.stack([p['b_shrink'] for p in nets])[:, :, None]
    As = jnp.stack([p['a_shrink'] for p in nets])[:, :, None]
    Wm = jnp.stack([jnp.stack(p['w_map']) for p in nets])
    Wm = Wm.transpose(0, 1, 2, 4, 5, 3).reshape(3, 4, S, 9 * S)
    Bm = jnp.stack([jnp.stack(p['b_map']) for p in nets])[..., None]
    Am = jnp.stack([jnp.stack(p['a_map']) for p in nets])[..., None]
    We = padd('w_expand', 0)[:, :, :, 0, 0]
    Be = padd('b_expand', 0)[:, :, None]
    Ae = padd('a_expand', 0, ones=True)[:, :, None]
    wdec = padd('w_deconv', 1)
    wdec = jnp.pad(wdec, ((0, 0), (0, 0), (0, 0), (0, 1), (0, 1)))
    o_i, c_i, ky_i, kx_i = _DFOLD
    e_i = np.arange(3)[:, None, None, None]
    Wd = wdec[e_i, o_i[None], c_i[None], ky_i[None], kx_i[None]]
    Wd = Wd.reshape(3, 48, 9 * D)
    Bd = jnp.tile(jnp.stack([p['b_deconv'] for p in nets]), (1, 16))[:, :, None]
    return (Wh, Bh, Ah, Ws, Bs, As, Wm, Bm, Am, We, Be, Ae, Wd, Bd)


def _run_fsrcnn(x3, sel, valid, packed, m5, m3):
    (Wh, Bh, Ah, Ws, Bs, As, Wm, Bm, Am, We, Be, Ae, Wd, Bd) = packed

    def e_map(shape):
        zeros = (0,) * (len(shape) - 1)
        return pl.BlockSpec((1,) + shape[1:],
                            lambda i, sel_ref, val_ref, z=zeros: (sel_ref[i],) + z)

    def full(shape):
        zeros = (0,) * len(shape)
        return pl.BlockSpec(shape, lambda i, sel_ref, val_ref, z=zeros: z)

    grid_spec = pltpu.PrefetchScalarGridSpec(
        num_scalar_prefetch=2,
        grid=(64,),
        in_specs=[
            pl.BlockSpec((1, 3, 1024), lambda i, s, v: (i, 0, 0)),
            full(m5.shape), full(m3.shape),
            e_map(Wh.shape), e_map(Bh.shape), e_map(Ah.shape),
            e_map(Ws.shape), e_map(Bs.shape), e_map(As.shape),
            e_map(Wm.shape), e_map(Bm.shape), e_map(Am.shape),
            e_map(We.shape), e_map(Be.shape), e_map(Ae.shape),
            e_map(Wd.shape), e_map(Bd.shape),
        ],
        out_specs=pl.BlockSpec((1, 48, 1024), lambda i, s, v: (i, 0, 0)),
    )
    return pl.pallas_call(
        _fsrcnn_body,
        grid_spec=grid_spec,
        out_shape=jax.ShapeDtypeStruct((64, 48, 1024), jnp.float32),
    )(sel, valid, x3, m5, m3, Wh, Bh, Ah, Ws, Bs, As, Wm, Bm, Am,
      We, Be, Ae, Wd, Bd)


def kernel(x, params):
    logits = _run_classifier(x, params['cls'])

    expert = jnp.argmax(logits, axis=-1).astype(jnp.int32)
    onehot = (expert[:, None] == jnp.arange(3, dtype=jnp.int32)).astype(jnp.int32)
    ranks = jnp.cumsum(onehot, axis=0)
    caps = jnp.asarray(CAPS, jnp.int32)
    myrank = jnp.sum(ranks * onehot, axis=1)
    valid = (myrank <= caps[expert]).astype(jnp.int32)
    counts = jnp.minimum(ranks[-1], caps)

    packed = _pack_expert_params(params['nets'])
    m5, m3 = _make_masks()
    x3 = x.reshape(64, 3, 1024)
    y48 = _run_fsrcnn(x3, expert, valid, packed, m5, m3)

    y = y48.reshape(64, 4, 4, 3, 32, 32)
    y = y.transpose(0, 3, 4, 1, 5, 2).reshape(64, 3, 128, 128)
    return y, counts


# speedup vs baseline: 4.2696x; 1.0968x over previous
import jax
import jax.numpy as jnp
import numpy as np
from jax.experimental import pallas as pl
from jax.experimental.pallas import tpu as pltpu

UP = 4
CAPS = (34, 38, 29)
D = 56
S = 12


def _classifier_body(x48_ref, w1_ref, b1_ref, w2_ref, b2_ref, w3_ref, b3_ref,
                     w4_ref, b4_ref, w5_ref, b5_ref, fcw_ref, fcb_ref,
                     logits_ref):
    def lrelu(h):
        return jnp.where(h >= 0, h, 0.1 * h)

    h = jnp.dot(x48_ref[...], w1_ref[...],
                preferred_element_type=jnp.float32) + b1_ref[...]
    h = lrelu(h)
    h = lrelu(jnp.dot(h, w2_ref[...],
                      preferred_element_type=jnp.float32) + b2_ref[...])
    h = lrelu(jnp.dot(h, w3_ref[...],
                      preferred_element_type=jnp.float32) + b3_ref[...])
    h = lrelu(jnp.dot(h, w4_ref[...],
                      preferred_element_type=jnp.float32) + b4_ref[...])
    h = jnp.dot(h, w5_ref[...],
                preferred_element_type=jnp.float32) + b5_ref[...]
    hm = jnp.mean(h.reshape(64, 64, 32), axis=1)
    logits_ref[...] = jnp.dot(hm, fcw_ref[...],
                              preferred_element_type=jnp.float32) + fcb_ref[...]


def _run_classifier(x, cls):
    x48 = x.reshape(64, 3, 8, 4, 8, 4).transpose(0, 2, 4, 3, 5, 1).reshape(4096, 48)
    w1 = cls['w1'].transpose(2, 3, 1, 0).reshape(48, 128)
    fcw = cls['fc_w'].T
    args = (x48, w1, cls['b1'][None, :], cls['w2'][:, :, 0, 0].T,
            cls['b2'][None, :], cls['w3'][:, :, 0, 0].T, cls['b3'][None, :],
            cls['w4'][:, :, 0, 0].T, cls['b4'][None, :],
            cls['w5'][:, :, 0, 0].T, cls['b5'][None, :], fcw,
            cls['fc_b'][None, :])
    return pl.pallas_call(
        _classifier_body,
        out_shape=jax.ShapeDtypeStruct((64, 3), jnp.float32),
    )(*args)


PB = 4


def _shift_stack(h, k, masks):
    r = k // 2
    pieces = []
    t = 0
    for dy in range(-r, r + 1):
        for dx in range(-r, r + 1):
            off = dy * 32 + dx
            s = jnp.roll(h, -off, axis=1) if off else h
            if dy or dx:
                s = s * masks[t][None, :]
            pieces.append(s)
            t += 1
    return jnp.concatenate(pieces, axis=0)


def _fsrcnn_body(sel_ref, val_ref, x3_ref, m5_ref, m3_ref,
                 wh_ref, bh_ref, ah_ref, ws_ref, bs_ref, as_ref,
                 wm_ref, bm_ref, am_ref, we_ref, be_ref, ae_ref,
                 wd_ref, bd_ref, out_ref):
    i = pl.program_id(0)
    m5 = m5_ref[...]
    m3 = m3_ref[...]
    m3b = m3.astype(jnp.bfloat16)

    def mm(w, im):
        return jnp.dot(w, im, preferred_element_type=jnp.float32)

    def prelu(h, b, a):
        h = h + b
        return jnp.where(h >= 0, h, a * h)

    for j in range(PB):
        p = i * PB + j
        e = sel_ref[p]
        x75 = _shift_stack(x3_ref[j], 5, m5)
        h = prelu(mm(wh_ref[e], x75), bh_ref[e], ah_ref[e])
        h = prelu(mm(ws_ref[e], h), bs_ref[e], as_ref[e])
        for l in range(4):
            im = _shift_stack(h, 3, m3)
            h = prelu(mm(wm_ref[e, l], im), bm_ref[e, l], am_ref[e, l])
        h = prelu(mm(we_ref[e], h), be_ref[e], ae_ref[e])
        im = _shift_stack(h.astype(jnp.bfloat16), 3, m3b)
        y = mm(wd_ref[e], im) + bd_ref[e]
        v = val_ref[p].astype(jnp.float32)
        out_ref[j] = y * v


def _make_masks():
    yy, xx = np.mgrid[0:32, 0:32]
    def mk(k):
        r = k // 2
        ms = []
        for dy in range(-r, r + 1):
            for dx in range(-r, r + 1):
                ok = ((yy + dy >= 0) & (yy + dy < 32) &
                      (xx + dx >= 0) & (xx + dx < 32))
                ms.append(ok.reshape(-1))
        return jnp.asarray(np.stack(ms).astype(np.float32))
    return mk(5), mk(3)


def _deconv_fold_idx():
    o_i = np.zeros((48, 9, D), np.int32)
    c_i = np.zeros((48, 9, D), np.int32)
    ky_i = np.full((48, 9, D), 9, np.int32)
    kx_i = np.full((48, 9, D), 9, np.int32)
    for ry in range(4):
        for rx in range(4):
            for o in range(3):
                ch = (ry * 4 + rx) * 3 + o
                o_i[ch] = o
                c_i[ch] = np.arange(D)[None, :]
                for dy in (-1, 0, 1):
                    for dx in (-1, 0, 1):
                        ky, kx = 4 * dy + 6 - ry, 4 * dx + 6 - rx
                        t = (dy + 1) * 3 + (dx + 1)
                        if 0 <= ky < 9 and 0 <= kx < 9:
                            ky_i[ch, t] = ky
                            kx_i[ch, t] = kx
    return o_i, c_i, ky_i, kx_i


_DFOLD = _deconv_fold_idx()


def _pack_expert_params(nets):
    def padd(name, axis, ones=False):
        cv = 1.0 if ones else 0.0
        outs = []
        for p in nets:
            w = p[name]
            d = w.shape[axis]
            pads = [(0, 0)] * w.ndim
            pads[axis] = (0, D - d)
            outs.append(jnp.pad(w, pads, constant_values=cv))
        return jnp.stack(outs)

    Wh = padd('w_head', 0).transpose(0, 1, 3, 4, 2).reshape(3, D, 75)
    Bh = padd('b_head', 0)[:, :, None]
    Ah = padd('a_head', 0, ones=True)[:, :, None]
    Ws = padd('w_shrink', 1)[:, :, :, 0, 0]
    Bs = jnp.stack([p['b_shrink'] for p in nets])[:, :, None]
    As = jnp.stack([p['a_shrink'] for p in nets])[:, :, None]
    Wm = jnp.stack([jnp.stack(p['w_map']) for p in nets])
    Wm = Wm.transpose(0, 1, 2, 4, 5, 3).reshape(3, 4, S, 9 * S)
    Bm = jnp.stack([jnp.stack(p['b_map']) for p in nets])[..., None]
    Am = jnp.stack([jnp.stack(p['a_map']) for p in nets])[..., None]
    We = padd('w_expand', 0)[:, :, :, 0, 0]
    Be = padd('b_expand', 0)[:, :, None]
    Ae = padd('a_expand', 0, ones=True)[:, :, None]
    wdec = padd('w_deconv', 1)
    wdec = jnp.pad(wdec, ((0, 0), (0, 0), (0, 0), (0, 1), (0, 1)))
    o_i, c_i, ky_i, kx_i = _DFOLD
    e_i = np.arange(3)[:, None, None, None]
    Wd = wdec[e_i, o_i[None], c_i[None], ky_i[None], kx_i[None]]
    Wd = Wd.reshape(3, 48, 9 * D).astype(jnp.bfloat16)
    Bd = jnp.tile(jnp.stack([p['b_deconv'] for p in nets]), (1, 16))[:, :, None]
    return (Wh, Bh, Ah, Ws, Bs, As, Wm, Bm, Am, We, Be, Ae, Wd, Bd)


def _run_fsrcnn(x3, sel, valid, packed, m5, m3):
    (Wh, Bh, Ah, Ws, Bs, As, Wm, Bm, Am, We, Be, Ae, Wd, Bd) = packed

    def full(shape):
        zeros = (0,) * len(shape)
        return pl.BlockSpec(shape, lambda i, sel_ref, val_ref, z=zeros: z)

    grid_spec = pltpu.PrefetchScalarGridSpec(
        num_scalar_prefetch=2,
        grid=(64 // PB,),
        in_specs=[
            pl.BlockSpec((PB, 3, 1024), lambda i, s, v: (i, 0, 0)),
            full(m5.shape), full(m3.shape),
            full(Wh.shape), full(Bh.shape), full(Ah.shape),
            full(Ws.shape), full(Bs.shape), full(As.shape),
            full(Wm.shape), full(Bm.shape), full(Am.shape),
            full(We.shape), full(Be.shape), full(Ae.shape),
            full(Wd.shape), full(Bd.shape),
        ],
        out_specs=pl.BlockSpec((PB, 48, 1024), lambda i, s, v: (i, 0, 0)),
    )
    return pl.pallas_call(
        _fsrcnn_body,
        grid_spec=grid_spec,
        out_shape=jax.ShapeDtypeStruct((64, 48, 1024), jnp.float32),
    )(sel, valid, x3, m5, m3, Wh, Bh, Ah, Ws, Bs, As, Wm, Bm, Am,
      We, Be, Ae, Wd, Bd)


def kernel(x, params):
    logits = _run_classifier(x, params['cls'])

    expert = jnp.argmax(logits, axis=-1).astype(jnp.int32)
    onehot = (expert[:, None] == jnp.arange(3, dtype=jnp.int32)).astype(jnp.int32)
    ranks = jnp.cumsum(onehot, axis=0)
    caps = jnp.asarray(CAPS, jnp.int32)
    myrank = jnp.sum(ranks * onehot, axis=1)
    valid = (myrank <= caps[expert]).astype(jnp.int32)
    counts = jnp.minimum(ranks[-1], caps)

    packed = _pack_expert_params(params['nets'])
    m5, m3 = _make_masks()
    x3 = x.reshape(64, 3, 1024)
    y48 = _run_fsrcnn(x3, expert, valid, packed, m5, m3)

    y = y48.reshape(64, 4, 4, 3, 32, 32)
    y = y.transpose(0, 3, 4, 1, 5, 2).reshape(64, 3, 128, 128)
    return y, counts
